# Initial kernel scaffold; baseline (speedup 1.0000x reference)
import jax
import jax.numpy as jnp
from jax.experimental import pallas as pl


def kernel(poi_embs, hg_poi_src, hg_poi_tar):
    raise NotImplementedError("write your pallas kernel here")


# speedup vs baseline: 1.5209x; 1.5209x over previous
import functools

import jax
import jax.numpy as jnp
from jax.experimental import pallas as pl
from jax.experimental.pallas import tpu as pltpu

_BM = 512


def _mm_body(a_ref, b_ref, o_ref, *, out_dtype):
    a = a_ref[...].astype(jnp.bfloat16)
    acc = jnp.dot(a, b_ref[...], preferred_element_type=jnp.float32)
    o_ref[...] = acc.astype(out_dtype)


def _mm(a, b, out_dtype):
    m, k = a.shape
    _, n = b.shape
    return pl.pallas_call(
        functools.partial(_mm_body, out_dtype=out_dtype),
        grid=(m // _BM,),
        in_specs=[
            pl.BlockSpec((_BM, k), lambda i: (i, 0)),
            pl.BlockSpec((k, n), lambda i: (0, 0)),
        ],
        out_specs=pl.BlockSpec((_BM, n), lambda i: (i, 0)),
        out_shape=jax.ShapeDtypeStruct((m, n), out_dtype),
        compiler_params=pltpu.CompilerParams(
            dimension_semantics=("parallel",),
        ),
    )(a, b)


def kernel(poi_embs, hg_poi_src, hg_poi_tar):
    embs_bf16 = poi_embs.astype(jnp.bfloat16)
    msg_tar = _mm(hg_poi_tar, embs_bf16, jnp.bfloat16)
    return _mm(hg_poi_src, msg_tar, jnp.float32)
